# baseline (speedup 1.0000x reference)
import functools

import jax
import jax.numpy as jnp
from jax import lax
from jax.experimental import pallas as pl
from jax.experimental.pallas import tpu as pltpu
from jax.experimental.pallas import tpu_sc as plsc

N = 10000
E = 320000
IN_F = 128
OUT_F = 16

NC = 2
NS = 16
NW = NC * NS
L = 16

N_PAD = 10240
ROWS_SC_TILE = N_PAD // NS
ROWS_W_TILE = N_PAD // NW

CPT = 80
N_CHUNKS = NW * CPT
E_PAD = N_CHUNKS * 128

NBUF = 8
DLAG = 2

_mesh = plsc.VectorSubcoreMesh(core_axis_name="c", subcore_axis_name="s")
_sc_params = pltpu.CompilerParams(use_tc_tiling_on_sc=False)
f32 = jnp.float32
i32 = jnp.int32


def _gather_ring(y_hbm, srcv, rowsv, gsems, ssems, fire, drain):
    def _buf(b):
        return rowsv.at[pl.ds(b * 128, 128)]

    def _gstart(j, b):
        return pltpu.async_copy(y_hbm.at[srcv.at[j]], _buf(b), gsems.at[b])

    for b in range(NBUF):
        _gstart(b, b)

    def outer(q, c):
        for b in range(NBUF):
            j = q * NBUF + b
            pltpu.make_async_copy(y_hbm.at[srcv.at[j]], _buf(b), gsems.at[b]).wait()
            fire(j, _buf(b), ssems.at[b])
            jd = j - DLAG
            bd = (b - DLAG) % NBUF

            @pl.when(jnp.logical_and(jd >= 0, jd + NBUF < CPT))
            def _():
                drain(jd, _buf(bd), ssems.at[bd])
                _gstart(jd + NBUF, bd)
        return c

    lax.fori_loop(0, CPT // NBUF, outer, 0)
    for b in range(NBUF):
        drain(CPT - NBUF + b, _buf(b), ssems.at[b])


def _nrsqrt(dd):
    ii = lax.bitcast_convert_type(dd, i32)
    ii = 0x5F3759DF - (ii >> 1)
    yv = lax.bitcast_convert_type(ii, f32)
    yv = yv * (1.5 - 0.5 * dd * yv * yv)
    yv = yv * (1.5 - 0.5 * dd * yv * yv)
    yv = yv * (1.5 - 0.5 * dd * yv * yv)
    return yv


def _mega_body(y_hbm, ed_hbm, ones_hbm, zeros_hbm, b_hbm,
               rst_hbm, p_hbm, da_hbm, db_hbm, t1_hbm, t2_hbm,
               srcv, dstv, onesv, rowsv, stagev, av, bv2, cv, nov, niv, bvv,
               accum, dega, degb, gsems, ssems, gsem):
    cid = lax.axis_index("c")
    sid = lax.axis_index("s")
    wid = sid * NC + cid
    r0 = sid * ROWS_SC_TILE
    w0 = wid * ROWS_W_TILE

    def gbar():
        plsc.subcore_barrier()

        @pl.when(sid == 0)
        def _():
            pltpu.semaphore_signal(gsem, 1, core_index=1 - cid)
            pl.semaphore_wait(gsem, 1)

        plsc.subcore_barrier()

    def zero_accum(ref):
        pltpu.sync_copy(zeros_hbm, stagev)
        pltpu.sync_copy(stagev, ref.at[pl.ds(r0, ROWS_SC_TILE)])

    def dump(ref, out):
        pltpu.sync_copy(ref.at[pl.ds(r0, ROWS_SC_TILE)], stagev)
        pltpu.sync_copy(stagev, out.at[cid].at[pl.ds(r0, ROWS_SC_TILE)])

    def load2(src2, dst_a, dst_b):
        pltpu.sync_copy(src2.at[0].at[pl.ds(w0, ROWS_W_TILE)], dst_a)
        pltpu.sync_copy(src2.at[1].at[pl.ds(w0, ROWS_W_TILE)], dst_b)

    zero_accum(accum)
    zero_accum(dega)
    zero_accum(degb)
    pltpu.sync_copy(ones_hbm, onesv)
    pltpu.sync_copy(b_hbm, bvv)
    c0 = wid * CPT
    pltpu.sync_copy(ed_hbm.at[0].at[pl.ds(c0, CPT)], srcv)
    pltpu.sync_copy(ed_hbm.at[1].at[pl.ds(c0, CPT)], dstv)
    plsc.subcore_barrier()

    def fire1(j, rows, ssem):
        pltpu.async_copy(rows, accum.at[dstv.at[j]], ssem, add=True)
        pltpu.async_copy(onesv, dega.at[srcv.at[j]], ssem, add=True)
        pltpu.async_copy(onesv, degb.at[dstv.at[j]], ssem, add=True)

    def drain1(j, rows, ssem):
        pltpu.make_async_copy(rows, accum.at[dstv.at[j]], ssem).wait()
        pltpu.make_async_copy(onesv, dega.at[srcv.at[j]], ssem).wait()
        pltpu.make_async_copy(onesv, degb.at[dstv.at[j]], ssem).wait()

    _gather_ring(y_hbm, srcv, rowsv, gsems, ssems, fire1, drain1)
    plsc.subcore_barrier()
    dump(accum, p_hbm)
    dump(dega, da_hbm)
    dump(degb, db_hbm)
    gbar()

    def addloop(i, c):
        cv[i] = av[i] + bv2[i]
        return c

    load2(p_hbm, av, bv2)
    lax.fori_loop(0, ROWS_W_TILE, addloop, 0)
    pltpu.sync_copy(cv, t1_hbm.at[pl.ds(w0, ROWS_W_TILE)])

    def normloop(out_ref):
        def body(i, c):
            out_ref[i] = _nrsqrt(jnp.maximum(av[i] + bv2[i], 1.0))
            return c
        lax.fori_loop(0, ROWS_W_TILE, body, 0)

    load2(da_hbm, av, bv2)
    normloop(nov)
    load2(db_hbm, av, bv2)
    normloop(niv)
    zero_accum(accum)
    gbar()

    def fire(j, rows, ssem):
        pltpu.async_copy(rows, accum.at[dstv.at[j]], ssem, add=True)

    def drain(j, rows, ssem):
        pltpu.make_async_copy(rows, accum.at[dstv.at[j]], ssem).wait()

    _gather_ring(t1_hbm, srcv, rowsv, gsems, ssems, fire, drain)
    plsc.subcore_barrier()
    dump(accum, p_hbm)
    gbar()

    load2(p_hbm, av, bv2)

    def scaleloop(i, c):
        cv[i] = (av[i] + bv2[i]) * nov[i]
        return c

    lax.fori_loop(0, ROWS_W_TILE, scaleloop, 0)
    pltpu.sync_copy(cv, t2_hbm.at[pl.ds(w0, ROWS_W_TILE)])
    zero_accum(accum)
    gbar()

    _gather_ring(t2_hbm, srcv, rowsv, gsems, ssems, fire, drain)
    plsc.subcore_barrier()
    dump(accum, p_hbm)
    gbar()

    load2(p_hbm, av, bv2)
    bvec = bvv[...]

    def finloop(i, c):
        cv[i] = (av[i] + bv2[i]) * niv[i] + bvec
        return c

    lax.fori_loop(0, ROWS_W_TILE, finloop, 0)
    pltpu.sync_copy(cv, rst_hbm.at[pl.ds(w0, ROWS_W_TILE)])


def _mega(y_p, ed, ones_in, zeros_in, b):
    sds = jax.ShapeDtypeStruct
    kern = pl.kernel(
        _mega_body,
        out_type=[
            sds((N_PAD, OUT_F), f32),
            sds((NC, N_PAD, OUT_F), f32),
            sds((NC, N_PAD, OUT_F), f32),
            sds((NC, N_PAD, OUT_F), f32),
            sds((N_PAD, OUT_F), f32),
            sds((N_PAD, OUT_F), f32),
        ],
        mesh=_mesh,
        compiler_params=_sc_params,
        scratch_types=[
            pltpu.VMEM((CPT, 128), i32),
            pltpu.VMEM((CPT, 128), i32),
            pltpu.VMEM((128, OUT_F), f32),
            pltpu.VMEM((NBUF * 128, OUT_F), f32),
            pltpu.VMEM((ROWS_SC_TILE, OUT_F), f32),
            pltpu.VMEM((ROWS_W_TILE, OUT_F), f32),
            pltpu.VMEM((ROWS_W_TILE, OUT_F), f32),
            pltpu.VMEM((ROWS_W_TILE, OUT_F), f32),
            pltpu.VMEM((ROWS_W_TILE, OUT_F), f32),
            pltpu.VMEM((ROWS_W_TILE, OUT_F), f32),
            pltpu.VMEM((OUT_F,), f32),
            pltpu.VMEM_SHARED((N_PAD, OUT_F), f32),
            pltpu.VMEM_SHARED((N_PAD, OUT_F), f32),
            pltpu.VMEM_SHARED((N_PAD, OUT_F), f32),
            pltpu.SemaphoreType.DMA((NBUF,)),
            pltpu.SemaphoreType.DMA((NBUF,)),
            pltpu.SemaphoreType.REGULAR,
        ],
    )
    return kern(y_p, ed, ones_in, zeros_in, b)


def _mm_body(x_ref, w_ref, o_ref):
    o_ref[...] = jnp.dot(x_ref[...], w_ref[...], preferred_element_type=f32)


def _matmul(x, W):
    return pl.pallas_call(
        _mm_body,
        grid=(N_PAD // 1280,),
        in_specs=[
            pl.BlockSpec((1280, IN_F), lambda i: (i, 0)),
            pl.BlockSpec((IN_F, OUT_F), lambda i: (0, 0)),
        ],
        out_specs=pl.BlockSpec((1280, OUT_F), lambda i: (i, 0)),
        out_shape=jax.ShapeDtypeStruct((N_PAD, OUT_F), f32),
    )(x, W)


def kernel(x, edge_index, W, b):
    pad = N + (jnp.arange(E_PAD - E, dtype=i32) % (N_PAD - N))
    ed = jnp.concatenate(
        [edge_index.astype(i32), jnp.broadcast_to(pad, (2, E_PAD - E))], axis=1
    ).reshape(2, N_CHUNKS, 128)

    y_p = _matmul(x, W)
    ones_in = jnp.ones((128, OUT_F), f32)
    zeros_in = jnp.zeros((ROWS_SC_TILE, OUT_F), f32)

    rst = _mega(y_p, ed, ones_in, zeros_in, b)[0]
    return rst[:N]


# speedup vs baseline: 1.0123x; 1.0123x over previous
import functools

import jax
import jax.numpy as jnp
from jax import lax
from jax.experimental import pallas as pl
from jax.experimental.pallas import tpu as pltpu
from jax.experimental.pallas import tpu_sc as plsc

N = 10000
E = 320000
IN_F = 128
OUT_F = 16

NC = 2
NS = 16
NW = NC * NS
L = 16

N_PAD = 10240
ROWS_SC_TILE = N_PAD // NS
ROWS_W_TILE = N_PAD // NW

CPT = 80
N_CHUNKS = NW * CPT
E_PAD = N_CHUNKS * 128

NBUF = 8
DLAG = 2

_mesh = plsc.VectorSubcoreMesh(core_axis_name="c", subcore_axis_name="s")
_sc_params = pltpu.CompilerParams(use_tc_tiling_on_sc=False)
f32 = jnp.float32
i32 = jnp.int32


def _gather_ring(y_hbm, srcv, rowsv, gsems, ssems, fire, drain):
    def _buf(b):
        return rowsv.at[pl.ds(b * 128, 128)]

    def _gstart(j, b):
        return pltpu.async_copy(y_hbm.at[srcv.at[j]], _buf(b), gsems.at[b])

    for b in range(NBUF):
        _gstart(b, b)

    def outer(q, c):
        for b in range(NBUF):
            j = q * NBUF + b
            pltpu.make_async_copy(y_hbm.at[srcv.at[j]], _buf(b), gsems.at[b]).wait()
            fire(j, _buf(b), ssems.at[b])
            jd = j - DLAG
            bd = (b - DLAG) % NBUF

            @pl.when(jnp.logical_and(jd >= 0, jd + NBUF < CPT))
            def _():
                drain(jd, _buf(bd), ssems.at[bd])
                _gstart(jd + NBUF, bd)
        return c

    lax.fori_loop(0, CPT // NBUF, outer, 0)
    for b in range(NBUF):
        drain(CPT - NBUF + b, _buf(b), ssems.at[b])


def _nrsqrt(dd):
    ii = lax.bitcast_convert_type(dd, i32)
    ii = 0x5F3759DF - (ii >> 1)
    yv = lax.bitcast_convert_type(ii, f32)
    yv = yv * (1.5 - 0.5 * dd * yv * yv)
    yv = yv * (1.5 - 0.5 * dd * yv * yv)
    yv = yv * (1.5 - 0.5 * dd * yv * yv)
    return yv


def _mega_body(y_hbm, ed_hbm, ones_hbm, zeros_hbm, b_hbm,
               rst_hbm, p_hbm, da_hbm, db_hbm, t1_hbm, t2_hbm,
               srcv, dstv, onesv, rowsv, stagev, av, bv2, cv, nov, niv, bvv,
               accum, dega, degb, gsems, ssems, gsem):
    cid = lax.axis_index("c")
    sid = lax.axis_index("s")
    wid = sid * NC + cid
    r0 = sid * ROWS_SC_TILE
    w0 = wid * ROWS_W_TILE

    def gbar():
        plsc.subcore_barrier()

        @pl.when(sid == 0)
        def _():
            pltpu.semaphore_signal(gsem, 1, core_index=1 - cid)
            pl.semaphore_wait(gsem, 1)

        plsc.subcore_barrier()

    def zero_accum(ref):
        pltpu.sync_copy(zeros_hbm, stagev)
        pltpu.sync_copy(stagev, ref.at[pl.ds(r0, ROWS_SC_TILE)])

    def dump(ref, out):
        pltpu.sync_copy(ref.at[pl.ds(r0, ROWS_SC_TILE)], stagev)
        pltpu.sync_copy(stagev, out.at[cid].at[pl.ds(r0, ROWS_SC_TILE)])

    def load2(src2, dst_a, dst_b):
        pltpu.sync_copy(src2.at[0].at[pl.ds(w0, ROWS_W_TILE)], dst_a)
        pltpu.sync_copy(src2.at[1].at[pl.ds(w0, ROWS_W_TILE)], dst_b)

    zero_accum(accum)
    zero_accum(dega)
    zero_accum(degb)
    pltpu.sync_copy(ones_hbm, onesv)
    pltpu.sync_copy(b_hbm, bvv)
    c0 = wid * CPT
    pltpu.sync_copy(ed_hbm.at[0].at[pl.ds(c0, CPT)], srcv)
    pltpu.sync_copy(ed_hbm.at[1].at[pl.ds(c0, CPT)], dstv)
    plsc.subcore_barrier()

    def fire1(j, rows, ssem):
        pltpu.async_copy(rows, accum.at[dstv.at[j]], ssem, add=True)
        pltpu.async_copy(onesv, dega.at[srcv.at[j]], ssem, add=True)
        pltpu.async_copy(onesv, degb.at[dstv.at[j]], ssem, add=True)

    def drain1(j, rows, ssem):
        pltpu.make_async_copy(rows, accum.at[dstv.at[j]], ssem).wait()
        pltpu.make_async_copy(onesv, dega.at[srcv.at[j]], ssem).wait()
        pltpu.make_async_copy(onesv, degb.at[dstv.at[j]], ssem).wait()

    _gather_ring(y_hbm, srcv, rowsv, gsems, ssems, fire1, drain1)
    plsc.subcore_barrier()
    dump(accum, p_hbm)
    dump(dega, da_hbm)
    dump(degb, db_hbm)
    gbar()

    UNR = 8

    def addloop(q, c):
        for k in range(UNR):
            i = q * UNR + k
            cv[i] = av[i] + bv2[i]
        return c

    load2(p_hbm, av, bv2)
    lax.fori_loop(0, ROWS_W_TILE // UNR, addloop, 0)
    pltpu.sync_copy(cv, t1_hbm.at[pl.ds(w0, ROWS_W_TILE)])

    def normloop(out_ref):
        def body(q, c):
            for k in range(UNR):
                i = q * UNR + k
                out_ref[i] = _nrsqrt(jnp.maximum(av[i] + bv2[i], 1.0))
            return c
        lax.fori_loop(0, ROWS_W_TILE // UNR, body, 0)

    load2(da_hbm, av, bv2)
    normloop(nov)
    load2(db_hbm, av, bv2)
    normloop(niv)
    zero_accum(accum)
    gbar()

    def fire(j, rows, ssem):
        pltpu.async_copy(rows, accum.at[dstv.at[j]], ssem, add=True)

    def drain(j, rows, ssem):
        pltpu.make_async_copy(rows, accum.at[dstv.at[j]], ssem).wait()

    _gather_ring(t1_hbm, srcv, rowsv, gsems, ssems, fire, drain)
    plsc.subcore_barrier()
    dump(accum, p_hbm)
    gbar()

    load2(p_hbm, av, bv2)

    def scaleloop(q, c):
        for k in range(UNR):
            i = q * UNR + k
            cv[i] = (av[i] + bv2[i]) * nov[i]
        return c

    lax.fori_loop(0, ROWS_W_TILE // UNR, scaleloop, 0)
    pltpu.sync_copy(cv, t2_hbm.at[pl.ds(w0, ROWS_W_TILE)])
    zero_accum(accum)
    gbar()

    _gather_ring(t2_hbm, srcv, rowsv, gsems, ssems, fire, drain)
    plsc.subcore_barrier()
    dump(accum, p_hbm)
    gbar()

    load2(p_hbm, av, bv2)
    bvec = bvv[...]

    def finloop(q, c):
        for k in range(UNR):
            i = q * UNR + k
            cv[i] = (av[i] + bv2[i]) * niv[i] + bvec
        return c

    lax.fori_loop(0, ROWS_W_TILE // UNR, finloop, 0)
    pltpu.sync_copy(cv, rst_hbm.at[pl.ds(w0, ROWS_W_TILE)])


def _mega(y_p, ed, ones_in, zeros_in, b):
    sds = jax.ShapeDtypeStruct
    kern = pl.kernel(
        _mega_body,
        out_type=[
            sds((N_PAD, OUT_F), f32),
            sds((NC, N_PAD, OUT_F), f32),
            sds((NC, N_PAD, OUT_F), f32),
            sds((NC, N_PAD, OUT_F), f32),
            sds((N_PAD, OUT_F), f32),
            sds((N_PAD, OUT_F), f32),
        ],
        mesh=_mesh,
        compiler_params=_sc_params,
        scratch_types=[
            pltpu.VMEM((CPT, 128), i32),
            pltpu.VMEM((CPT, 128), i32),
            pltpu.VMEM((128, OUT_F), f32),
            pltpu.VMEM((NBUF * 128, OUT_F), f32),
            pltpu.VMEM((ROWS_SC_TILE, OUT_F), f32),
            pltpu.VMEM((ROWS_W_TILE, OUT_F), f32),
            pltpu.VMEM((ROWS_W_TILE, OUT_F), f32),
            pltpu.VMEM((ROWS_W_TILE, OUT_F), f32),
            pltpu.VMEM((ROWS_W_TILE, OUT_F), f32),
            pltpu.VMEM((ROWS_W_TILE, OUT_F), f32),
            pltpu.VMEM((OUT_F,), f32),
            pltpu.VMEM_SHARED((N_PAD, OUT_F), f32),
            pltpu.VMEM_SHARED((N_PAD, OUT_F), f32),
            pltpu.VMEM_SHARED((N_PAD, OUT_F), f32),
            pltpu.SemaphoreType.DMA((NBUF,)),
            pltpu.SemaphoreType.DMA((NBUF,)),
            pltpu.SemaphoreType.REGULAR,
        ],
    )
    return kern(y_p, ed, ones_in, zeros_in, b)


def _mm_body(x_ref, w_ref, o_ref):
    o_ref[...] = jnp.dot(x_ref[...], w_ref[...], preferred_element_type=f32)


def _matmul(x, W):
    return pl.pallas_call(
        _mm_body,
        grid=(N_PAD // 1280,),
        in_specs=[
            pl.BlockSpec((1280, IN_F), lambda i: (i, 0)),
            pl.BlockSpec((IN_F, OUT_F), lambda i: (0, 0)),
        ],
        out_specs=pl.BlockSpec((1280, OUT_F), lambda i: (i, 0)),
        out_shape=jax.ShapeDtypeStruct((N_PAD, OUT_F), f32),
    )(x, W)


def kernel(x, edge_index, W, b):
    pad = N + (jnp.arange(E_PAD - E, dtype=i32) % (N_PAD - N))
    ed = jnp.concatenate(
        [edge_index.astype(i32), jnp.broadcast_to(pad, (2, E_PAD - E))], axis=1
    ).reshape(2, N_CHUNKS, 128)

    y_p = _matmul(x, W)
    ones_in = jnp.ones((128, OUT_F), f32)
    zeros_in = jnp.zeros((ROWS_SC_TILE, OUT_F), f32)

    rst = _mega(y_p, ed, ones_in, zeros_in, b)[0]
    return rst[:N]


# speedup vs baseline: 1.0476x; 1.0349x over previous
import functools

import jax
import jax.numpy as jnp
from jax import lax
from jax.experimental import pallas as pl
from jax.experimental.pallas import tpu as pltpu
from jax.experimental.pallas import tpu_sc as plsc

N = 10000
E = 320000
IN_F = 128
OUT_F = 16

NC = 2
NS = 16
NW = NC * NS
L = 16

N_PAD = 10240
ROWS_SC_TILE = N_PAD // NS
ROWS_W_TILE = N_PAD // NW
FLAT = N_PAD * OUT_F
FLAT_TILE = FLAT // NW

CPT = 80
N_CHUNKS = NW * CPT
E_PAD = N_CHUNKS * 128

_mesh = plsc.VectorSubcoreMesh(core_axis_name="c", subcore_axis_name="s")
_sc_params = pltpu.CompilerParams(use_tc_tiling_on_sc=False)
f32 = jnp.float32
i32 = jnp.int32


def _wid():
    return lax.axis_index("s") * NC + lax.axis_index("c")


NBUF = 8
DLAG = 2


def _gather_ring(y_hbm, srcv, rowsv, gsems, ssems, fire, drain):
    def _buf(b):
        return rowsv.at[pl.ds(b * 128, 128)]

    def _gstart(j, b):
        return pltpu.async_copy(y_hbm.at[srcv.at[j]], _buf(b), gsems.at[b])

    for b in range(NBUF):
        _gstart(b, b)

    def outer(q, c):
        for b in range(NBUF):
            j = q * NBUF + b
            pltpu.make_async_copy(y_hbm.at[srcv.at[j]], _buf(b), gsems.at[b]).wait()
            fire(j, _buf(b), ssems.at[b])
            jd = j - DLAG
            bd = (b - DLAG) % NBUF

            @pl.when(jnp.logical_and(jd >= 0, jd + NBUF < CPT))
            def _():
                drain(jd, _buf(bd), ssems.at[bd])
                _gstart(jd + NBUF, bd)
        return c

    lax.fori_loop(0, CPT // NBUF, outer, 0)
    for b in range(NBUF):
        drain(CPT - NBUF + b, _buf(b), ssems.at[b])


def _round_body_deg(y_hbm, ed_hbm, ones_hbm, zeros_hbm,
                    p_hbm, da_hbm, db_hbm,
                    srcv, dstv, onesv, rowsv, stagev, accum, dega, degb,
                    gsems, ssems):
    cid = lax.axis_index("c")
    sid = lax.axis_index("s")
    wid = sid * NC + cid
    r0 = sid * ROWS_SC_TILE
    pltpu.sync_copy(zeros_hbm, stagev)
    pltpu.sync_copy(stagev, accum.at[pl.ds(r0, ROWS_SC_TILE)])
    pltpu.sync_copy(stagev, dega.at[pl.ds(r0, ROWS_SC_TILE)])
    pltpu.sync_copy(stagev, degb.at[pl.ds(r0, ROWS_SC_TILE)])
    pltpu.sync_copy(ones_hbm, onesv)
    c0 = wid * CPT
    pltpu.sync_copy(ed_hbm.at[0].at[pl.ds(c0, CPT)], srcv)
    pltpu.sync_copy(ed_hbm.at[1].at[pl.ds(c0, CPT)], dstv)
    plsc.subcore_barrier()

    def fire(j, rows, ssem):
        pltpu.async_copy(rows, accum.at[dstv.at[j]], ssem, add=True)
        pltpu.async_copy(onesv, dega.at[srcv.at[j]], ssem, add=True)
        pltpu.async_copy(onesv, degb.at[dstv.at[j]], ssem, add=True)

    def drain(j, rows, ssem):
        pltpu.make_async_copy(rows, accum.at[dstv.at[j]], ssem).wait()
        pltpu.make_async_copy(onesv, dega.at[srcv.at[j]], ssem).wait()
        pltpu.make_async_copy(onesv, degb.at[dstv.at[j]], ssem).wait()

    _gather_ring(y_hbm, srcv, rowsv, gsems, ssems, fire, drain)
    plsc.subcore_barrier()
    pltpu.sync_copy(accum.at[pl.ds(r0, ROWS_SC_TILE)], stagev)
    pltpu.sync_copy(stagev, p_hbm.at[cid].at[pl.ds(r0, ROWS_SC_TILE)])
    pltpu.sync_copy(dega.at[pl.ds(r0, ROWS_SC_TILE)], stagev)
    pltpu.sync_copy(stagev, da_hbm.at[cid].at[pl.ds(r0, ROWS_SC_TILE)])
    pltpu.sync_copy(degb.at[pl.ds(r0, ROWS_SC_TILE)], stagev)
    pltpu.sync_copy(stagev, db_hbm.at[cid].at[pl.ds(r0, ROWS_SC_TILE)])


def _round_body(y_hbm, ed_hbm, zeros_hbm, p_hbm,
                srcv, dstv, rowsv, stagev, accum, gsems, ssems):
    cid = lax.axis_index("c")
    sid = lax.axis_index("s")
    wid = sid * NC + cid
    r0 = sid * ROWS_SC_TILE
    pltpu.sync_copy(zeros_hbm, stagev)
    pltpu.sync_copy(stagev, accum.at[pl.ds(r0, ROWS_SC_TILE)])
    c0 = wid * CPT
    pltpu.sync_copy(ed_hbm.at[0].at[pl.ds(c0, CPT)], srcv)
    pltpu.sync_copy(ed_hbm.at[1].at[pl.ds(c0, CPT)], dstv)
    plsc.subcore_barrier()

    def fire(j, rows, ssem):
        pltpu.async_copy(rows, accum.at[dstv.at[j]], ssem, add=True)

    def drain(j, rows, ssem):
        pltpu.make_async_copy(rows, accum.at[dstv.at[j]], ssem).wait()

    _gather_ring(y_hbm, srcv, rowsv, gsems, ssems, fire, drain)
    plsc.subcore_barrier()
    pltpu.sync_copy(accum.at[pl.ds(r0, ROWS_SC_TILE)], stagev)
    pltpu.sync_copy(stagev, p_hbm.at[cid].at[pl.ds(r0, ROWS_SC_TILE)])


def _round_deg(y_p, ed, ones_in, zeros_in):
    kern = pl.kernel(
        _round_body_deg,
        out_type=[jax.ShapeDtypeStruct((NC, N_PAD, OUT_F), f32)] * 3,
        mesh=_mesh,
        compiler_params=_sc_params,
        scratch_types=[
            pltpu.VMEM((CPT, 128), i32),
            pltpu.VMEM((CPT, 128), i32),
            pltpu.VMEM((128, OUT_F), f32),
            pltpu.VMEM((NBUF * 128, OUT_F), f32),
            pltpu.VMEM((ROWS_SC_TILE, OUT_F), f32),
            pltpu.VMEM_SHARED((N_PAD, OUT_F), f32),
            pltpu.VMEM_SHARED((N_PAD, OUT_F), f32),
            pltpu.VMEM_SHARED((N_PAD, OUT_F), f32),
            pltpu.SemaphoreType.DMA((NBUF,)),
            pltpu.SemaphoreType.DMA((NBUF,)),
        ],
    )
    return kern(y_p, ed, ones_in, zeros_in)


def _round(t_in, ed, zeros_in):
    kern = pl.kernel(
        _round_body,
        out_type=jax.ShapeDtypeStruct((NC, N_PAD, OUT_F), f32),
        mesh=_mesh,
        compiler_params=_sc_params,
        scratch_types=[
            pltpu.VMEM((CPT, 128), i32),
            pltpu.VMEM((CPT, 128), i32),
            pltpu.VMEM((NBUF * 128, OUT_F), f32),
            pltpu.VMEM((ROWS_SC_TILE, OUT_F), f32),
            pltpu.VMEM_SHARED((N_PAD, OUT_F), f32),
            pltpu.SemaphoreType.DMA((NBUF,)),
            pltpu.SemaphoreType.DMA((NBUF,)),
        ],
    )
    return kern(t_in, ed, zeros_in)


TCR = FLAT // 128


def _tc_combine_norm_body(p_ref, da_ref, db_ref, t1_ref, on_ref, in_ref):
    t1_ref[...] = p_ref[0] + p_ref[1]
    on_ref[...] = lax.rsqrt(jnp.maximum(da_ref[0] + da_ref[1], 1.0))
    in_ref[...] = lax.rsqrt(jnp.maximum(db_ref[0] + db_ref[1], 1.0))


def _combine_norm(p1, dA, dB):
    return pl.pallas_call(
        _tc_combine_norm_body,
        out_shape=[jax.ShapeDtypeStruct((TCR, 128), f32)] * 3,
    )(p1, dA, dB)


def _tc_combine_scale_body(p_ref, n_ref, o_ref):
    o_ref[...] = (p_ref[0] + p_ref[1]) * n_ref[...]


def _combine_scale(p2, onorm):
    return pl.pallas_call(
        _tc_combine_scale_body,
        out_shape=jax.ShapeDtypeStruct((TCR, 128), f32),
    )(p2, onorm)


def _tc_combine_scale_bias_body(p_ref, n_ref, b_ref, o_ref):
    o_ref[...] = (p_ref[0] + p_ref[1]) * n_ref[...] + b_ref[...]


def _combine_scale_bias(p3, inorm, b_tile):
    return pl.pallas_call(
        _tc_combine_scale_bias_body,
        out_shape=jax.ShapeDtypeStruct((TCR, 128), f32),
    )(p3, inorm, b_tile)


def _mm_body(x_ref, w_ref, o_ref):
    o_ref[...] = jnp.dot(x_ref[...], w_ref[...], preferred_element_type=f32)


def _matmul(x, W):
    return pl.pallas_call(
        _mm_body,
        grid=(N_PAD // 1280,),
        in_specs=[
            pl.BlockSpec((1280, IN_F), lambda i: (i, 0)),
            pl.BlockSpec((IN_F, OUT_F), lambda i: (0, 0)),
        ],
        out_specs=pl.BlockSpec((1280, OUT_F), lambda i: (i, 0)),
        out_shape=jax.ShapeDtypeStruct((N_PAD, OUT_F), f32),
    )(x, W)


def kernel(x, edge_index, W, b):
    pad = N + (jnp.arange(E_PAD - E, dtype=i32) % (N_PAD - N))
    ed = jnp.concatenate(
        [edge_index.astype(i32), jnp.broadcast_to(pad, (2, E_PAD - E))], axis=1
    ).reshape(2, N_CHUNKS, 128)

    y_p = _matmul(x, W)

    ones_in = jnp.ones((128, OUT_F), f32)
    zeros_in = jnp.zeros((ROWS_SC_TILE, OUT_F), f32)
    b_tile = jnp.tile(b, 128 // OUT_F)

    p1, dA, dB = _round_deg(y_p, ed, ones_in, zeros_in)
    t1, onorm, inorm = _combine_norm(
        p1.reshape(NC, TCR, 128), dA.reshape(NC, TCR, 128), dB.reshape(NC, TCR, 128))

    p2 = _round(t1.reshape(N_PAD, OUT_F), ed, zeros_in)
    t2 = _combine_scale(p2.reshape(NC, TCR, 128), onorm)

    p3 = _round(t2.reshape(N_PAD, OUT_F), ed, zeros_in)
    rst = _combine_scale_bias(p3.reshape(NC, TCR, 128), inorm, b_tile)

    return rst.reshape(N_PAD, OUT_F)[:N]
